# Initial kernel scaffold; baseline (speedup 1.0000x reference)
import functools

import jax
import jax.numpy as jnp
from jax import lax
from jax.experimental import pallas as pl
from jax.experimental.pallas import tpu as pltpu
from jax.experimental.pallas import tpu_sc as plsc

B, C, H, W = 2, 8, 512, 512
OH, OW = 1024, 1024
BC = B * C
NPIX = OH * OW
NW = 32
PER_W = NPIX // NW
CHUNK = 128
GROUPS = CHUNK // 16
OUTBUF = 2048
CPF = OUTBUF // CHUNK
NFLUSH = PER_W // OUTBUF

_mesh = plsc.VectorSubcoreMesh(core_axis_name="c", subcore_axis_name="s")


@functools.partial(
    pl.kernel,
    mesh=_mesh,
    out_type=jax.ShapeDtypeStruct((BC, NPIX), jnp.float32),
    scratch_types=[
        pltpu.VMEM((CHUNK,), jnp.float32),
        pltpu.VMEM((CHUNK,), jnp.float32),
        pltpu.VMEM((CHUNK,), jnp.float32),
        pltpu.VMEM((CHUNK,), jnp.float32),
        pltpu.VMEM((CHUNK,), jnp.int32),
        pltpu.VMEM((CHUNK,), jnp.int32),
        pltpu.VMEM((CHUNK,), jnp.int32),
        pltpu.VMEM((CHUNK,), jnp.int32),
        pltpu.VMEM((CHUNK, BC), jnp.float32),
        pltpu.VMEM((CHUNK, BC), jnp.float32),
        pltpu.VMEM((CHUNK, BC), jnp.float32),
        pltpu.VMEM((CHUNK, BC), jnp.float32),
        pltpu.VMEM((BC, OUTBUF), jnp.float32),
        pltpu.SemaphoreType.DMA,
    ],
)
def _unresample_sc(xt_hbm, sx_hbm, sy_hbm, out_hbm,
                   sxv, syv, wxv, wyv, i00, i01, i10, i11,
                   v00, v01, v10, v11, outv, sem):
    wid = lax.axis_index("s") * 2 + lax.axis_index("c")
    base_w = wid * PER_W

    @pl.loop(0, NFLUSH)
    def _(fb):
        fbase = base_w + fb * OUTBUF

        @pl.loop(0, CPF)
        def _(cj):
            base = fbase + cj * CHUNK
            pltpu.sync_copy(sx_hbm.at[pl.ds(base, CHUNK)], sxv)
            pltpu.sync_copy(sy_hbm.at[pl.ds(base, CHUNK)], syv)

            @pl.loop(0, GROUPS)
            def _(g):
                s = pl.ds(g * 16, 16)
                sxg = sxv[s]
                syg = syv[s]
                x0 = jnp.minimum(sxg.astype(jnp.int32), W - 2)
                y0 = jnp.minimum(syg.astype(jnp.int32), H - 2)
                wxv[s] = sxg - x0.astype(jnp.float32)
                wyv[s] = syg - y0.astype(jnp.float32)
                b00 = y0 * W + x0
                i00[s] = b00
                i01[s] = b00 + 1
                i10[s] = b00 + W
                i11[s] = b00 + (W + 1)

            c0 = pltpu.async_copy(xt_hbm.at[i00], v00, sem)
            c1 = pltpu.async_copy(xt_hbm.at[i01], v01, sem)
            c2 = pltpu.async_copy(xt_hbm.at[i10], v10, sem)
            c3 = pltpu.async_copy(xt_hbm.at[i11], v11, sem)
            c0.wait()
            c1.wait()
            c2.wait()
            c3.wait()

            obase = cj * CHUNK

            @pl.loop(0, GROUPS)
            def _(g):
                s = pl.ds(g * 16, 16)
                wxg = wxv[s]
                wyg = wyv[s]
                w00 = (1.0 - wyg) * (1.0 - wxg)
                w01 = (1.0 - wyg) * wxg
                w10 = wyg * (1.0 - wxg)
                w11 = wyg * wxg
                rows = g * 16 + lax.iota(jnp.int32, 16)
                for bc in range(BC):
                    col = jnp.full((16,), bc, jnp.int32)
                    t00 = plsc.load_gather(v00, [rows, col])
                    t01 = plsc.load_gather(v01, [rows, col])
                    t10 = plsc.load_gather(v10, [rows, col])
                    t11 = plsc.load_gather(v11, [rows, col])
                    outv[bc, pl.ds(obase + g * 16, 16)] = (
                        w00 * t00 + w01 * t01 + w10 * t10 + w11 * t11)

        for bc in range(BC):
            pltpu.sync_copy(outv.at[bc],
                            out_hbm.at[bc, pl.ds(fbase, OUTBUF)])


def kernel(x, sample_map):
    xt = jnp.transpose(x, (2, 3, 0, 1)).reshape(H * W, BC)
    sm = sample_map.reshape(NPIX, 2)
    out = _unresample_sc(xt, sm[:, 0], sm[:, 1])
    return out.reshape(B, C, OH, OW)


# speedup vs baseline: 49.9653x; 49.9653x over previous
import dataclasses
import functools

import jax
import jax.numpy as jnp
from jax import lax
from jax.experimental import pallas as pl
from jax.experimental.pallas import tpu as pltpu
from jax.experimental.pallas import tpu_sc as plsc

B, C, H, W = 2, 8, 512, 512
OH, OW = 1024, 1024
BC = B * C
NPIX = OH * OW
NW = 32
PER_W = NPIX // NW
CHUNK = 128
GROUPS = CHUNK // 16
OUTBUF = 2048
CPF = OUTBUF // CHUNK
NFLUSH = PER_W // OUTBUF

_mesh = plsc.VectorSubcoreMesh(core_axis_name="c", subcore_axis_name="s")

_cp = pltpu.CompilerParams()
if "needs_layout_passes" in pltpu.CompilerParams.__dataclass_fields__:
    _cp = dataclasses.replace(_cp, needs_layout_passes=False)
if "use_tc_tiling_on_sc" in pltpu.CompilerParams.__dataclass_fields__:
    _cp = dataclasses.replace(_cp, use_tc_tiling_on_sc=False)


@functools.partial(
    pl.kernel,
    mesh=_mesh,
    compiler_params=_cp,
    out_type=jax.ShapeDtypeStruct((B, C, OH, OW), jnp.float32),
    scratch_types=[
        pltpu.VMEM((CHUNK,), jnp.float32),
        pltpu.VMEM((CHUNK,), jnp.float32),
        pltpu.VMEM((CHUNK,), jnp.float32),
        pltpu.VMEM((CHUNK,), jnp.float32),
        pltpu.VMEM((CHUNK,), jnp.int32),
        pltpu.VMEM((CHUNK,), jnp.int32),
        pltpu.VMEM((CHUNK,), jnp.int32),
        pltpu.VMEM((CHUNK,), jnp.int32),
        pltpu.VMEM((CHUNK, BC), jnp.float32),
        pltpu.VMEM((CHUNK, BC), jnp.float32),
        pltpu.VMEM((CHUNK, BC), jnp.float32),
        pltpu.VMEM((CHUNK, BC), jnp.float32),
        pltpu.VMEM((BC, OUTBUF), jnp.float32),
        pltpu.SemaphoreType.DMA,
    ],
)
def _unresample_sc(xt_hbm, sx_hbm, sy_hbm, out_hbm,
                   sxv, syv, wxv, wyv, i00, i01, i10, i11,
                   v00, v01, v10, v11, outv, sem):
    wid = lax.axis_index("s") * 2 + lax.axis_index("c")
    base_w = wid * PER_W

    @pl.loop(0, NFLUSH)
    def _(fb):
        fbase = base_w + fb * OUTBUF

        @pl.loop(0, CPF)
        def _(cj):
            base = fbase + cj * CHUNK
            pltpu.sync_copy(sx_hbm.at[pl.ds(base, CHUNK)], sxv)
            pltpu.sync_copy(sy_hbm.at[pl.ds(base, CHUNK)], syv)

            @pl.loop(0, GROUPS)
            def _(g):
                s = pl.ds(g * 16, 16)
                sxg = sxv[s]
                syg = syv[s]
                x0 = jnp.minimum(sxg.astype(jnp.int32), W - 2)
                y0 = jnp.minimum(syg.astype(jnp.int32), H - 2)
                wxv[s] = sxg - x0.astype(jnp.float32)
                wyv[s] = syg - y0.astype(jnp.float32)
                b00 = y0 * W + x0
                i00[s] = b00
                i01[s] = b00 + 1
                i10[s] = b00 + W
                i11[s] = b00 + (W + 1)

            c0 = pltpu.async_copy(xt_hbm.at[i00], v00, sem)
            c1 = pltpu.async_copy(xt_hbm.at[i01], v01, sem)
            c2 = pltpu.async_copy(xt_hbm.at[i10], v10, sem)
            c3 = pltpu.async_copy(xt_hbm.at[i11], v11, sem)
            c0.wait()
            c1.wait()
            c2.wait()
            c3.wait()

            obase = cj * CHUNK

            @pl.loop(0, GROUPS)
            def _(g):
                s = pl.ds(g * 16, 16)
                wxg = wxv[s]
                wyg = wyv[s]
                w00 = (1.0 - wyg) * (1.0 - wxg)
                w01 = (1.0 - wyg) * wxg
                w10 = wyg * (1.0 - wxg)
                w11 = wyg * wxg
                rows = g * 16 + lax.iota(jnp.int32, 16)
                for bc in range(BC):
                    col = jnp.full((16,), bc, jnp.int32)
                    t00 = plsc.load_gather(v00, [rows, col])
                    t01 = plsc.load_gather(v01, [rows, col])
                    t10 = plsc.load_gather(v10, [rows, col])
                    t11 = plsc.load_gather(v11, [rows, col])
                    outv[bc, pl.ds(obase + g * 16, 16)] = (
                        w00 * t00 + w01 * t01 + w10 * t10 + w11 * t11)

        rowbase = wid * (PER_W // OW) + fb * (OUTBUF // OW)
        for bc in range(BC):
            b, c = bc // C, bc % C
            pltpu.sync_copy(outv.at[bc, pl.ds(0, OW)],
                            out_hbm.at[b, c, rowbase])
            pltpu.sync_copy(outv.at[bc, pl.ds(OW, OW)],
                            out_hbm.at[b, c, rowbase + 1])


def kernel(x, sample_map):
    xt = jnp.transpose(x, (2, 3, 0, 1)).reshape(H * W, BC)
    sm = sample_map.reshape(NPIX, 2)
    return _unresample_sc(xt, sm[:, 0], sm[:, 1])
